# baseline (speedup 1.0000x reference)
import functools

import jax
import jax.numpy as jnp
from jax.experimental import pallas as pl
from jax.experimental.pallas import tpu as pltpu

_P = 32


def _mlp_body(x_ref, w0_ref, b0_ref, w1_ref, b1_ref, w2_ref, b2_ref, o_ref):
    h = jnp.dot(x_ref[...], w0_ref[...], preferred_element_type=jnp.float32)
    h = jnp.maximum(h + b0_ref[...], 0.0)
    h = jnp.dot(h, w1_ref[...], preferred_element_type=jnp.float32)
    h = jnp.maximum(h + b1_ref[...], 0.0)
    o_ref[...] = (
        jnp.dot(h, w2_ref[...], preferred_element_type=jnp.float32)
        + b2_ref[...]
    )


@jax.jit
def _forward(x, w0, b0, w1, b1, w2, b2):
    B, in_f = x.shape
    P = _P

    eye = jnp.eye(P, dtype=jnp.float32)
    w0p = jnp.kron(eye, w0.T)
    w1p = jnp.kron(eye, w1.T)
    w2p = jnp.kron(eye, w2.T)
    b0p = jnp.tile(b0[:, 0], P)[None]
    b1p = jnp.tile(b1[:, 0], P)[None]
    b2p = jnp.tile(b2[:, 0], P)[None]

    rows = pl.cdiv(B, P)
    if rows * P != B:
        x = jnp.pad(x, ((0, rows * P - B), (0, 0)))
    x2 = x.reshape(rows, in_f * P)

    tile_m = min(4096, ((rows + 7) // 8) * 8)
    num_tiles = pl.cdiv(rows, tile_m)
    padded_rows = num_tiles * tile_m
    if padded_rows != rows:
        x2 = jnp.pad(x2, ((0, padded_rows - rows), (0, 0)))

    const_map = lambda i: (0, 0)
    out = pl.pallas_call(
        _mlp_body,
        out_shape=jax.ShapeDtypeStruct((padded_rows, P), jnp.float32),
        grid=(num_tiles,),
        in_specs=[
            pl.BlockSpec((tile_m, in_f * P), lambda i: (i, 0)),
            pl.BlockSpec((in_f * P, 7 * P), const_map),
            pl.BlockSpec((1, 7 * P), const_map),
            pl.BlockSpec((7 * P, 7 * P), const_map),
            pl.BlockSpec((1, 7 * P), const_map),
            pl.BlockSpec((7 * P, P), const_map),
            pl.BlockSpec((1, P), const_map),
        ],
        out_specs=pl.BlockSpec((tile_m, P), lambda i: (i, 0)),
        compiler_params=pltpu.CompilerParams(
            dimension_semantics=("parallel",),
        ),
    )(x2, w0p, b0p, w1p, b1p, w2p, b2p)

    return out.reshape(padded_rows * P, 1)[:B]


def kernel(x, w0, b0, w1, b1, w2, b2):
    return _forward(x, w0, b0, w1, b1, w2, b2)


# speedup vs baseline: 15.4874x; 15.4874x over previous
import functools

import jax
import jax.numpy as jnp
from jax.experimental import pallas as pl
from jax.experimental.pallas import tpu as pltpu

_TILE_B = 131072


def _mlp_body(x_ref, w0_ref, b0_ref, w1_ref, b1_ref, w2_ref, b2_ref, o_ref):
    bf = jnp.bfloat16
    x16 = x_ref[...].astype(bf)
    h = jnp.dot(w0_ref[...].astype(bf), x16, preferred_element_type=jnp.float32)
    h = jnp.maximum(h.astype(bf) + b0_ref[...].astype(bf), 0)
    h = jnp.dot(w1_ref[...].astype(bf), h, preferred_element_type=jnp.float32)
    h = jnp.maximum(h.astype(bf) + b1_ref[...].astype(bf), 0)
    out = jnp.dot(w2_ref[...].astype(bf), h, preferred_element_type=jnp.float32)
    o_ref[...] = out + b2_ref[...]


@jax.jit
def _forward(x, w0, b0, w1, b1, w2, b2):
    B, in_f = x.shape

    x_t = x.T

    num_tiles = pl.cdiv(B, _TILE_B)
    tile_b = min(_TILE_B, ((B + num_tiles * 128 - 1) // (num_tiles * 128)) * 128)
    padded_b = num_tiles * tile_b
    if padded_b != B:
        x_t = jnp.pad(x_t, ((0, 0), (0, padded_b - B)))

    const_map = lambda i: (0, 0)
    out = pl.pallas_call(
        _mlp_body,
        out_shape=jax.ShapeDtypeStruct((1, padded_b), jnp.float32),
        grid=(num_tiles,),
        in_specs=[
            pl.BlockSpec((in_f, tile_b), lambda i: (0, i)),
            pl.BlockSpec((7, in_f), const_map),
            pl.BlockSpec((7, 1), const_map),
            pl.BlockSpec((7, 7), const_map),
            pl.BlockSpec((7, 1), const_map),
            pl.BlockSpec((1, 7), const_map),
            pl.BlockSpec((1, 1), const_map),
        ],
        out_specs=pl.BlockSpec((1, tile_b), lambda i: (0, i)),
        compiler_params=pltpu.CompilerParams(
            dimension_semantics=("parallel",),
        ),
    )(x_t, w0, b0, w1, b1, w2, b2)

    return out[:, :B].T


def kernel(x, w0, b0, w1, b1, w2, b2):
    return _forward(x, w0, b0, w1, b1, w2, b2)
